# baseline (speedup 1.0000x reference)
import functools

import jax
import jax.numpy as jnp
from jax import lax
from jax.experimental import pallas as pl
from jax.experimental.pallas import tpu as pltpu
from jax.experimental.pallas import tpu_sc as plsc

N_CLASSES = 100000
EMBED_DIM = 64
BATCH = 16384

NC = 2
NS = 16
LANES = 16
NW = NC * NS
B_PER_W = BATCH // NW
CHUNK = 128
N_CHUNKS = B_PER_W // CHUNK

RB = 12544
T_GRID = 4
PACKED_ROWS = T_GRID * RB
SPLIT = PACKED_ROWS
KB = 4096


def _pack_body(lo_ref, hi_ref, o_ref):
    o_ref[:, 0:EMBED_DIM] = lo_ref[...].T
    o_ref[:, EMBED_DIM : 2 * EMBED_DIM] = hi_ref[...].T


def _pack_table(embed_t):
    return pl.pallas_call(
        _pack_body,
        grid=(T_GRID,),
        in_specs=[
            pl.BlockSpec((EMBED_DIM, RB), lambda j: (0, j)),
            pl.BlockSpec((EMBED_DIM, RB), lambda j: (0, j + T_GRID)),
        ],
        out_specs=pl.BlockSpec((RB, 2 * EMBED_DIM), lambda j: (j, 0)),
        out_shape=jax.ShapeDtypeStruct((PACKED_ROWS, 2 * EMBED_DIM),
                                       jnp.float32),
    )(embed_t, embed_t)


def _gather_kernel(cls_hbm, packed_hbm, mid_hbm, idx_v, q_v, rows_v, sem):
    wid = lax.axis_index("s") * NC + lax.axis_index("c")
    base = wid * B_PER_W

    pltpu.sync_copy(cls_hbm.at[pl.ds(base, B_PER_W)], idx_v)

    gathers = []
    for g in range(B_PER_W // LANES):
        v = idx_v[pl.ds(g * LANES, LANES)]
        v = jnp.where(v == 0, N_CLASSES - 1, v - 1)
        q_v[pl.ds(g * LANES, LANES)] = jnp.where(v >= SPLIT, v - SPLIT, v)
        if g % (CHUNK // LANES) == CHUNK // LANES - 1:
            j = g // (CHUNK // LANES)
            gathers.append(
                pltpu.async_copy(
                    packed_hbm.at[q_v.at[pl.ds(j * CHUNK, CHUNK)]],
                    rows_v.at[pl.ds(j * CHUNK, CHUNK)],
                    sem,
                )
            )
    for c in gathers:
        c.wait()

    pltpu.sync_copy(rows_v, mid_hbm.at[pl.ds(base, B_PER_W)])


def _unpack_body(cls_ref, m_ref, o_ref):
    iv = cls_ref[...]
    idx = jnp.where(iv == 0, N_CLASSES - 1, iv - 1)
    hi = idx >= SPLIT
    xt = m_ref[...].T
    o_ref[...] = jnp.where(hi[None, :], xt[EMBED_DIM:, :], xt[:EMBED_DIM, :])


@jax.jit
def kernel(embed, cls):
    packed = _pack_table(embed.T)

    mesh = plsc.VectorSubcoreMesh(core_axis_name="c", subcore_axis_name="s")
    run = functools.partial(
        pl.kernel,
        out_type=jax.ShapeDtypeStruct((BATCH, 2 * EMBED_DIM), jnp.float32),
        mesh=mesh,
        scratch_types=[
            pltpu.VMEM((B_PER_W,), jnp.int32),
            pltpu.VMEM((B_PER_W,), jnp.int32),
            pltpu.VMEM((B_PER_W, 2 * EMBED_DIM), jnp.float32),
            pltpu.SemaphoreType.DMA,
        ],
        compiler_params=pltpu.CompilerParams(
            use_tc_tiling_on_sc=True, needs_layout_passes=False
        ),
    )(_gather_kernel)
    mid = run(cls, packed)

    out_t = pl.pallas_call(
        _unpack_body,
        grid=(BATCH // KB,),
        in_specs=[
            pl.BlockSpec((KB,), lambda j: (j,)),
            pl.BlockSpec((KB, 2 * EMBED_DIM), lambda j: (j, 0)),
        ],
        out_specs=pl.BlockSpec((EMBED_DIM, KB), lambda j: (0, j)),
        out_shape=jax.ShapeDtypeStruct((EMBED_DIM, BATCH), jnp.float32),
    )(cls, mid)
    return out_t.T


# speedup vs baseline: 1.0145x; 1.0145x over previous
import functools

import jax
import jax.numpy as jnp
from jax import lax
from jax.experimental import pallas as pl
from jax.experimental.pallas import tpu as pltpu
from jax.experimental.pallas import tpu_sc as plsc

N_CLASSES = 100000
EMBED_DIM = 64
BATCH = 16384

NC = 2
NS = 16
LANES = 16
NW = NC * NS
B_PER_W = BATCH // NW
CHUNK = 128
N_CHUNKS = B_PER_W // CHUNK

RB = 12544
T_GRID = 4
PACKED_ROWS = T_GRID * RB
SPLIT = PACKED_ROWS
KB = 8192


def _pack_body(lo_ref, hi_ref, o_ref):
    o_ref[:, 0:EMBED_DIM] = lo_ref[...].T
    o_ref[:, EMBED_DIM : 2 * EMBED_DIM] = hi_ref[...].T


def _pack_table(embed_t):
    return pl.pallas_call(
        _pack_body,
        grid=(T_GRID,),
        in_specs=[
            pl.BlockSpec((EMBED_DIM, RB), lambda j: (0, j)),
            pl.BlockSpec((EMBED_DIM, RB), lambda j: (0, j + T_GRID)),
        ],
        out_specs=pl.BlockSpec((RB, 2 * EMBED_DIM), lambda j: (j, 0)),
        out_shape=jax.ShapeDtypeStruct((PACKED_ROWS, 2 * EMBED_DIM),
                                       jnp.float32),
    )(embed_t, embed_t)


def _gather_kernel(cls_hbm, packed_hbm, mid_hbm, idx_v, q_v, rows_v, sem):
    wid = lax.axis_index("s") * NC + lax.axis_index("c")
    base = wid * B_PER_W

    pltpu.sync_copy(cls_hbm.at[pl.ds(base, B_PER_W)], idx_v)

    gathers = []
    for g in range(B_PER_W // LANES):
        v = idx_v[pl.ds(g * LANES, LANES)]
        v = jnp.where(v == 0, N_CLASSES - 1, v - 1)
        q_v[pl.ds(g * LANES, LANES)] = jnp.where(v >= SPLIT, v - SPLIT, v)
        if g % (CHUNK // LANES) == CHUNK // LANES - 1:
            j = g // (CHUNK // LANES)
            gathers.append(
                pltpu.async_copy(
                    packed_hbm.at[q_v.at[pl.ds(j * CHUNK, CHUNK)]],
                    rows_v.at[pl.ds(j * CHUNK, CHUNK)],
                    sem,
                )
            )
    for c in gathers:
        c.wait()

    pltpu.sync_copy(rows_v, mid_hbm.at[pl.ds(base, B_PER_W)])


def _unpack_body(cls_ref, m_ref, o_ref):
    iv = cls_ref[...]
    idx = jnp.where(iv == 0, N_CLASSES - 1, iv - 1)
    hi = idx >= SPLIT
    xt = m_ref[...].T
    o_ref[...] = jnp.where(hi[None, :], xt[EMBED_DIM:, :], xt[:EMBED_DIM, :])


@jax.jit
def kernel(embed, cls):
    packed = _pack_table(embed.T)

    mesh = plsc.VectorSubcoreMesh(core_axis_name="c", subcore_axis_name="s")
    run = functools.partial(
        pl.kernel,
        out_type=jax.ShapeDtypeStruct((BATCH, 2 * EMBED_DIM), jnp.float32),
        mesh=mesh,
        scratch_types=[
            pltpu.VMEM((B_PER_W,), jnp.int32),
            pltpu.VMEM((B_PER_W,), jnp.int32),
            pltpu.VMEM((B_PER_W, 2 * EMBED_DIM), jnp.float32),
            pltpu.SemaphoreType.DMA,
        ],
        compiler_params=pltpu.CompilerParams(
            use_tc_tiling_on_sc=True, needs_layout_passes=False
        ),
    )(_gather_kernel)
    mid = run(cls, packed)

    out_t = pl.pallas_call(
        _unpack_body,
        grid=(BATCH // KB,),
        in_specs=[
            pl.BlockSpec((KB,), lambda j: (j,)),
            pl.BlockSpec((KB, 2 * EMBED_DIM), lambda j: (j, 0)),
        ],
        out_specs=pl.BlockSpec((EMBED_DIM, KB), lambda j: (0, j)),
        out_shape=jax.ShapeDtypeStruct((EMBED_DIM, BATCH), jnp.float32),
    )(cls, mid)
    return out_t.T
